# Initial kernel scaffold; baseline (speedup 1.0000x reference)
import jax
import jax.numpy as jnp
from jax.experimental import pallas as pl


def kernel(sinputs, tinputs, slabels, W0, b0, W1, b1, gat0_W, gat0_a, gat1_W, gat1_a, Wc, bc, Wd, bd):
    raise NotImplementedError("write your pallas kernel here")


# speedup vs baseline: 5.3441x; 5.3441x over previous
import jax
import jax.numpy as jnp
from jax.experimental import pallas as pl

_D = 3
_B = 512
_IN = 512
_H1, _H2 = 512, 256
_G = 128
_NH = 4
_K = 10
_ALPHA = 0.2
_MARGIN = 1.0
_NCLS = 10
_N = (_D + 1) * _B
_F = _NH * _G
_RB = 256
_NEG = -9e15


def _mlp_body(x_ref, w0_ref, b0_ref, w1_ref, b1_ref, o_ref):
    h = jnp.dot(x_ref[...], w0_ref[...], preferred_element_type=jnp.float32)
    h = jnp.maximum(h + b0_ref[...], 0.0)
    h = jnp.dot(h, w1_ref[...], preferred_element_type=jnp.float32)
    o_ref[...] = jnp.maximum(h + b1_ref[...], 0.0)


def _topk_body(hb_ref, hf_ref, idx_ref):
    i = pl.program_id(0)
    hb = hb_ref[...]
    hf = hf_ref[...]
    xxb = jnp.sum(hb * hb, axis=1, keepdims=True)
    xxf = jnp.sum(hf * hf, axis=1)
    g = jax.lax.dot_general(hb, hf, (((1,), (1,)), ((), ())),
                            preferred_element_type=jnp.float32)
    d2 = jnp.maximum(xxb + xxf[None, :] - 2.0 * g, 0.0)
    cols = jax.lax.broadcasted_iota(jnp.int32, d2.shape, 1)
    rows = jax.lax.broadcasted_iota(jnp.int32, d2.shape, 0) + i * _RB
    neg = jnp.where(rows == cols, -1e12, -d2)
    picks = []
    for _ in range(_K):
        m = jnp.max(neg, axis=1, keepdims=True)
        it = jnp.min(jnp.where(neg == m, cols, _N), axis=1, keepdims=True)
        picks.append(it)
        neg = jnp.where(cols == it, -jnp.float32(jnp.inf), neg)
    picks.append(jnp.zeros((_RB, 16 - _K), jnp.int32))
    idx_ref[...] = jnp.concatenate(picks, axis=1)


def _hff_body(x_ref, w_ref, a_ref, h_ref, f_ref):
    h = jnp.dot(x_ref[...], w_ref[...], preferred_element_type=jnp.float32)
    h_ref[...] = h
    f_ref[...] = jnp.dot(h, a_ref[...], preferred_element_type=jnp.float32)


def _gat_body(fb_ref, ft_ref, idx_ref, h_ref, o_ref):
    i = pl.program_id(0)
    cols = jax.lax.broadcasted_iota(jnp.int32, (_RB, _N), 1)
    rows = jax.lax.broadcasted_iota(jnp.int32, (_RB, _N), 0) + i * _RB
    idxb = idx_ref[...]
    mask = cols == rows
    for t in range(_K):
        mask = mask | (cols == idxb[:, t:t + 1])
    for c in range(_NH):
        f1 = fb_ref[:, c:c + 1]
        f2 = ft_ref[c + _NH:c + _NH + 1, :]
        e = f1 + f2
        e = jnp.where(e >= 0.0, e, _ALPHA * e)
        e = jnp.where(mask, e, _NEG)
        m = jnp.max(e, axis=1, keepdims=True)
        p = jnp.where(mask, jnp.exp(e - m), 0.0)
        s = jnp.sum(p, axis=1, keepdims=True)
        att = p / s
        o = jnp.dot(att, h_ref[:, c * _G:(c + 1) * _G],
                    preferred_element_type=jnp.float32)
        o_ref[:, c * _G:(c + 1) * _G] = jnp.where(o > 0.0, o, jnp.exp(o) - 1.0)


def _log_softmax(x):
    m = jnp.max(x, axis=1, keepdims=True)
    return (x - m) - jnp.log(jnp.sum(jnp.exp(x - m), axis=1, keepdims=True))


def _heads_body(x_ref, lab_ref, labt_ref, wc_ref, bc_ref, wd_ref, bd_ref,
                lp_ref, sd_ref, td_ref, tl_ref):
    gt = x_ref[_D * _B:, :]
    rt = jnp.maximum(gt, 0.0)
    cols = jax.lax.broadcasted_iota(jnp.int32, (_B, _B), 1)
    rows = jax.lax.broadcasted_iota(jnp.int32, (_B, _B), 0)
    eye = rows == cols
    for d in range(_D):
        g = x_ref[d * _B:(d + 1) * _B, :]
        r = jnp.maximum(g, 0.0)
        lg = jnp.dot(r, wc_ref[...], preferred_element_type=jnp.float32) + bc_ref[...]
        lp_ref[d] = _log_softmax(lg)
        wd = wd_ref[d]
        bd = bd_ref[d:d + 1, :]
        sd_ref[d] = _log_softmax(
            jnp.dot(r, wd, preferred_element_type=jnp.float32) + bd)
        td_ref[d] = _log_softmax(
            jnp.dot(rt, wd, preferred_element_type=jnp.float32) + bd)
        nrm = jnp.sqrt(jnp.sum(g * g, axis=1, keepdims=True))
        gn = g / jnp.maximum(nrm, 1e-12)
        gram = jax.lax.dot_general(gn, gn, (((1,), (1,)), ((), ())),
                                   preferred_element_type=jnp.float32)
        diag = jnp.where(eye, gram, 0.0)
        xxc = jnp.sum(diag, axis=1, keepdims=True)
        xxr = jnp.sum(diag, axis=0, keepdims=True)
        dist = jnp.sqrt(jnp.maximum(xxc + xxr - 2.0 * gram, 0.0) + 1e-12)
        lr = lab_ref[d:d + 1, :]
        lc = labt_ref[:, d:d + 1]
        same = lc == lr
        pos_mask = same & (~eye)
        neg_mask = ~same
        pv = jnp.where(pos_mask, dist, -1.0)
        pm = jnp.max(pv, axis=1, keepdims=True)
        pidx = jnp.min(jnp.where(pv == pm, cols, _B), axis=1, keepdims=True)
        nv = jnp.where(neg_mask, dist, 1e12)
        nm = jnp.min(nv, axis=1, keepdims=True)
        nidx = jnp.min(jnp.where(nv == nm, cols, _B), axis=1, keepdims=True)
        pos_d = jnp.sum(jnp.where(cols == pidx, dist, 0.0), axis=1)
        neg_d = jnp.sum(jnp.where(cols == nidx, dist, 0.0), axis=1)
        hard = (neg_d - pos_d < _MARGIN).astype(jnp.float32)
        hinge = jnp.maximum(_MARGIN + pos_d - neg_d, 0.0)
        loss = jnp.sum(hinge * hard) / jnp.maximum(jnp.sum(hard), 1.0)
        tl_ref[d:d + 1, :] = jnp.full((1, 128), loss, jnp.float32)


def _build_A(a):
    a2 = a[:, :, 0]
    A = jnp.zeros((_F, 2 * _NH), jnp.float32)
    for c in range(_NH):
        A = A.at[c * _G:(c + 1) * _G, c].set(a2[c, :_G])
        A = A.at[c * _G:(c + 1) * _G, _NH + c].set(a2[c, _G:])
    return A


def _gat_layer(x, idx, gat_W, gat_a, in_dim):
    Wcat = jnp.transpose(gat_W, (1, 0, 2)).reshape(in_dim, _F)
    A = _build_A(gat_a)
    h_all, f = pl.pallas_call(
        _hff_body,
        grid=(4,),
        in_specs=[
            pl.BlockSpec((_N // 4, in_dim), lambda i: (i, 0)),
            pl.BlockSpec((in_dim, _F), lambda i: (0, 0)),
            pl.BlockSpec((_F, 2 * _NH), lambda i: (0, 0)),
        ],
        out_specs=[
            pl.BlockSpec((_N // 4, _F), lambda i: (i, 0)),
            pl.BlockSpec((_N // 4, 2 * _NH), lambda i: (i, 0)),
        ],
        out_shape=[
            jax.ShapeDtypeStruct((_N, _F), jnp.float32),
            jax.ShapeDtypeStruct((_N, 2 * _NH), jnp.float32),
        ],
    )(x, Wcat, A)
    ft = f.T
    return pl.pallas_call(
        _gat_body,
        grid=(_N // _RB,),
        in_specs=[
            pl.BlockSpec((_RB, 2 * _NH), lambda i: (i, 0)),
            pl.BlockSpec((2 * _NH, _N), lambda i: (0, 0)),
            pl.BlockSpec((_RB, 16), lambda i: (i, 0)),
            pl.BlockSpec((_N, _F), lambda i: (0, 0)),
        ],
        out_specs=pl.BlockSpec((_RB, _F), lambda i: (i, 0)),
        out_shape=jax.ShapeDtypeStruct((_N, _F), jnp.float32),
    )(f, ft, idx, h_all)


def kernel(sinputs, tinputs, slabels, W0, b0, W1, b1,
           gat0_W, gat0_a, gat1_W, gat1_a, Wc, bc, Wd, bd):
    x_in = jnp.concatenate([sinputs.reshape(_D * _B, _IN), tinputs], axis=0)
    h2 = pl.pallas_call(
        _mlp_body,
        grid=(4,),
        in_specs=[
            pl.BlockSpec((_N // 4, _IN), lambda i: (i, 0)),
            pl.BlockSpec((_IN, _H1), lambda i: (0, 0)),
            pl.BlockSpec((1, _H1), lambda i: (0, 0)),
            pl.BlockSpec((_H1, _H2), lambda i: (0, 0)),
            pl.BlockSpec((1, _H2), lambda i: (0, 0)),
        ],
        out_specs=pl.BlockSpec((_N // 4, _H2), lambda i: (i, 0)),
        out_shape=jax.ShapeDtypeStruct((_N, _H2), jnp.float32),
    )(x_in, W0, b0.reshape(1, _H1), W1, b1.reshape(1, _H2))

    idx = pl.pallas_call(
        _topk_body,
        grid=(_N // _RB,),
        in_specs=[
            pl.BlockSpec((_RB, _H2), lambda i: (i, 0)),
            pl.BlockSpec((_N, _H2), lambda i: (0, 0)),
        ],
        out_specs=pl.BlockSpec((_RB, 16), lambda i: (i, 0)),
        out_shape=jax.ShapeDtypeStruct((_N, 16), jnp.int32),
    )(h2, h2)

    x1 = _gat_layer(h2, idx, gat0_W, gat0_a, _H2)
    x2 = _gat_layer(x1, idx, gat1_W, gat1_a, _F)

    lab = slabels.astype(jnp.int32)
    labt = lab.T
    lp, sd, td, tl = pl.pallas_call(
        _heads_body,
        out_shape=[
            jax.ShapeDtypeStruct((_D, _B, _NCLS), jnp.float32),
            jax.ShapeDtypeStruct((_D, _B, 2), jnp.float32),
            jax.ShapeDtypeStruct((_D, _B, 2), jnp.float32),
            jax.ShapeDtypeStruct((_D, 128), jnp.float32),
        ],
    )(x2, lab, labt, Wc, bc.reshape(1, _NCLS), Wd, bd)
    return lp, sd, td, tl[:, 0]


# speedup vs baseline: 7.0480x; 1.3188x over previous
import jax
import jax.numpy as jnp
from jax.experimental import pallas as pl

_D = 3
_B = 512
_IN = 512
_H1, _H2 = 512, 256
_G = 128
_NH = 4
_K = 10
_ALPHA = 0.2
_MARGIN = 1.0
_NCLS = 10
_N = (_D + 1) * _B
_F = _NH * _G
_RB = 256
_NEG = -9e15


def _mlp_body(x_ref, w0_ref, b0_ref, w1_ref, b1_ref, o_ref):
    h = jnp.dot(x_ref[...], w0_ref[...], preferred_element_type=jnp.float32)
    h = jnp.maximum(h + b0_ref[...], 0.0)
    h = jnp.dot(h, w1_ref[...], preferred_element_type=jnp.float32)
    o_ref[...] = jnp.maximum(h + b1_ref[...], 0.0)


def _topk_body(hb_ref, hf_ref, idx_ref, mask_ref):
    i = pl.program_id(0)
    hb = hb_ref[...]
    hf = hf_ref[...]
    xxb = jnp.sum(hb * hb, axis=1, keepdims=True)
    xxf = jnp.sum(hf * hf, axis=1)
    g = jax.lax.dot_general(hb, hf, (((1,), (1,)), ((), ())),
                            preferred_element_type=jnp.float32)
    d2 = jnp.maximum(xxb + xxf[None, :] - 2.0 * g, 0.0)
    cols = jax.lax.broadcasted_iota(jnp.int32, d2.shape, 1)
    rows = jax.lax.broadcasted_iota(jnp.int32, d2.shape, 0) + i * _RB
    neg = jnp.where(rows == cols, -1e12, -d2)
    picks = []
    for _ in range(_K):
        m = jnp.max(neg, axis=1, keepdims=True)
        it = jnp.min(jnp.where(neg == m, cols, _N), axis=1, keepdims=True)
        picks.append(it)
        neg = jnp.where(cols == it, -jnp.float32(jnp.inf), neg)
    picks.append(jnp.zeros((_RB, 16 - _K), jnp.int32))
    idx_ref[...] = jnp.concatenate(picks, axis=1)
    mask_ref[...] = jnp.where((neg == -jnp.float32(jnp.inf)) | (rows == cols),
                              1.0, 0.0).astype(jnp.float32)


def _hff_body(x_ref, w_ref, a_ref, h_ref, f_ref):
    h = jnp.dot(x_ref[...], w_ref[...], preferred_element_type=jnp.float32)
    h_ref[...] = h
    f_ref[...] = jnp.dot(h, a_ref[...], preferred_element_type=jnp.float32)


def _gat_body(fb_ref, ft_ref, mask_ref, h_ref, o_ref):
    maskf = mask_ref[...]
    for c in range(_NH):
        f1 = fb_ref[:, c:c + 1]
        f2 = ft_ref[c + _NH:c + _NH + 1, :]
        z = f1 + f2
        p = maskf * jnp.exp(jnp.maximum(z, _ALPHA * z))
        s = jnp.sum(p, axis=1, keepdims=True)
        att = p * (1.0 / s)
        o = jnp.dot(att, h_ref[:, c * _G:(c + 1) * _G],
                    preferred_element_type=jnp.float32)
        o_ref[:, c * _G:(c + 1) * _G] = jnp.where(o > 0.0, o, jnp.exp(o) - 1.0)


def _log_softmax(x):
    m = jnp.max(x, axis=1, keepdims=True)
    return (x - m) - jnp.log(jnp.sum(jnp.exp(x - m), axis=1, keepdims=True))


def _heads_body(x_ref, lab_ref, labt_ref, wc_ref, bc_ref, wd_ref, bd_ref,
                lp_ref, sd_ref, td_ref, tl_ref):
    gt = x_ref[_D * _B:, :]
    rt = jnp.maximum(gt, 0.0)
    cols = jax.lax.broadcasted_iota(jnp.int32, (_B, _B), 1)
    rows = jax.lax.broadcasted_iota(jnp.int32, (_B, _B), 0)
    eye = rows == cols
    for d in range(_D):
        g = x_ref[d * _B:(d + 1) * _B, :]
        r = jnp.maximum(g, 0.0)
        lg = jnp.dot(r, wc_ref[...], preferred_element_type=jnp.float32) + bc_ref[...]
        lp_ref[d] = _log_softmax(lg)
        wd = wd_ref[d]
        bd = bd_ref[d:d + 1, :]
        sd_ref[d] = _log_softmax(
            jnp.dot(r, wd, preferred_element_type=jnp.float32) + bd)
        td_ref[d] = _log_softmax(
            jnp.dot(rt, wd, preferred_element_type=jnp.float32) + bd)
        nrm = jnp.sqrt(jnp.sum(g * g, axis=1, keepdims=True))
        gn = g / jnp.maximum(nrm, 1e-12)
        gram = jax.lax.dot_general(gn, gn, (((1,), (1,)), ((), ())),
                                   preferred_element_type=jnp.float32)
        diag = jnp.where(eye, gram, 0.0)
        xxc = jnp.sum(diag, axis=1, keepdims=True)
        xxr = jnp.sum(diag, axis=0, keepdims=True)
        dist = jnp.sqrt(jnp.maximum(xxc + xxr - 2.0 * gram, 0.0) + 1e-12)
        lr = lab_ref[d:d + 1, :]
        lc = labt_ref[:, d:d + 1]
        same = lc == lr
        pos_mask = same & (~eye)
        neg_mask = ~same
        pv = jnp.where(pos_mask, dist, -1.0)
        pm = jnp.max(pv, axis=1, keepdims=True)
        pidx = jnp.min(jnp.where(pv == pm, cols, _B), axis=1, keepdims=True)
        nv = jnp.where(neg_mask, dist, 1e12)
        nm = jnp.min(nv, axis=1, keepdims=True)
        nidx = jnp.min(jnp.where(nv == nm, cols, _B), axis=1, keepdims=True)
        pos_d = jnp.sum(jnp.where(cols == pidx, dist, 0.0), axis=1)
        neg_d = jnp.sum(jnp.where(cols == nidx, dist, 0.0), axis=1)
        hard = (neg_d - pos_d < _MARGIN).astype(jnp.float32)
        hinge = jnp.maximum(_MARGIN + pos_d - neg_d, 0.0)
        loss = jnp.sum(hinge * hard) / jnp.maximum(jnp.sum(hard), 1.0)
        tl_ref[d:d + 1, :] = jnp.full((1, 128), loss, jnp.float32)


def _build_A(a):
    a2 = a[:, :, 0]
    A = jnp.zeros((_F, 2 * _NH), jnp.float32)
    for c in range(_NH):
        A = A.at[c * _G:(c + 1) * _G, c].set(a2[c, :_G])
        A = A.at[c * _G:(c + 1) * _G, _NH + c].set(a2[c, _G:])
    return A


def _gat_layer(x, maskf, gat_W, gat_a, in_dim):
    Wcat = jnp.transpose(gat_W, (1, 0, 2)).reshape(in_dim, _F)
    A = _build_A(gat_a)
    h_all, f = pl.pallas_call(
        _hff_body,
        grid=(4,),
        in_specs=[
            pl.BlockSpec((_N // 4, in_dim), lambda i: (i, 0)),
            pl.BlockSpec((in_dim, _F), lambda i: (0, 0)),
            pl.BlockSpec((_F, 2 * _NH), lambda i: (0, 0)),
        ],
        out_specs=[
            pl.BlockSpec((_N // 4, _F), lambda i: (i, 0)),
            pl.BlockSpec((_N // 4, 2 * _NH), lambda i: (i, 0)),
        ],
        out_shape=[
            jax.ShapeDtypeStruct((_N, _F), jnp.float32),
            jax.ShapeDtypeStruct((_N, 2 * _NH), jnp.float32),
        ],
    )(x, Wcat, A)
    ft = f.T
    return pl.pallas_call(
        _gat_body,
        grid=(_N // _RB,),
        in_specs=[
            pl.BlockSpec((_RB, 2 * _NH), lambda i: (i, 0)),
            pl.BlockSpec((2 * _NH, _N), lambda i: (0, 0)),
            pl.BlockSpec((_RB, _N), lambda i: (i, 0)),
            pl.BlockSpec((_N, _F), lambda i: (0, 0)),
        ],
        out_specs=pl.BlockSpec((_RB, _F), lambda i: (i, 0)),
        out_shape=jax.ShapeDtypeStruct((_N, _F), jnp.float32),
    )(f, ft, maskf, h_all)


def kernel(sinputs, tinputs, slabels, W0, b0, W1, b1,
           gat0_W, gat0_a, gat1_W, gat1_a, Wc, bc, Wd, bd):
    x_in = jnp.concatenate([sinputs.reshape(_D * _B, _IN), tinputs], axis=0)
    h2 = pl.pallas_call(
        _mlp_body,
        grid=(4,),
        in_specs=[
            pl.BlockSpec((_N // 4, _IN), lambda i: (i, 0)),
            pl.BlockSpec((_IN, _H1), lambda i: (0, 0)),
            pl.BlockSpec((1, _H1), lambda i: (0, 0)),
            pl.BlockSpec((_H1, _H2), lambda i: (0, 0)),
            pl.BlockSpec((1, _H2), lambda i: (0, 0)),
        ],
        out_specs=pl.BlockSpec((_N // 4, _H2), lambda i: (i, 0)),
        out_shape=jax.ShapeDtypeStruct((_N, _H2), jnp.float32),
    )(x_in, W0, b0.reshape(1, _H1), W1, b1.reshape(1, _H2))

    idx, maskf = pl.pallas_call(
        _topk_body,
        grid=(_N // _RB,),
        in_specs=[
            pl.BlockSpec((_RB, _H2), lambda i: (i, 0)),
            pl.BlockSpec((_N, _H2), lambda i: (0, 0)),
        ],
        out_specs=[
            pl.BlockSpec((_RB, 16), lambda i: (i, 0)),
            pl.BlockSpec((_RB, _N), lambda i: (i, 0)),
        ],
        out_shape=[
            jax.ShapeDtypeStruct((_N, 16), jnp.int32),
            jax.ShapeDtypeStruct((_N, _N), jnp.float32),
        ],
    )(h2, h2)
    del idx

    x1 = _gat_layer(h2, maskf, gat0_W, gat0_a, _H2)
    x2 = _gat_layer(x1, maskf, gat1_W, gat1_a, _F)

    lab = slabels.astype(jnp.int32)
    labt = lab.T
    lp, sd, td, tl = pl.pallas_call(
        _heads_body,
        out_shape=[
            jax.ShapeDtypeStruct((_D, _B, _NCLS), jnp.float32),
            jax.ShapeDtypeStruct((_D, _B, 2), jnp.float32),
            jax.ShapeDtypeStruct((_D, _B, 2), jnp.float32),
            jax.ShapeDtypeStruct((_D, 128), jnp.float32),
        ],
    )(x2, lab, labt, Wc, bc.reshape(1, _NCLS), Wd, bd)
    return lp, sd, td, tl[:, 0]
